# baseline (speedup 1.0000x reference)
import jax
import jax.numpy as jnp
from jax.experimental import pallas as pl
from jax.experimental.pallas import tpu as pltpu

MARGIN = 1.0
BLK = 256


def _loss_body(xk_ref, lc_ref, lr_ref, out_ref,
               xs_ref, nrow_ref, accd_ref, acco_ref, sem):
    k = pl.program_id(0)
    nb = pl.num_programs(0)
    n_total = nb * BLK
    dim = xk_ref.shape[1]

    cp = pltpu.make_async_copy(xk_ref, xs_ref.at[pl.ds(k * BLK, BLK), :], sem)
    cp.start()

    @pl.when(k == 0)
    def _init():
        accd_ref[...] = jnp.zeros_like(accd_ref)
        acco_ref[...] = jnp.zeros_like(acco_ref)

    xk = xk_ref[...]
    nk = jax.lax.dot_general(
        jnp.ones((1, dim), jnp.float32), xk * xk,
        (((1,), (1,)), ((), ())),
        preferred_element_type=jnp.float32)
    nrow_ref[:, pl.ds(k * BLK, BLK)] = nk

    lrk = lr_ref[:, pl.ds(k * BLK, BLK)]

    def tile(xi, ni, li, acc_ref):
        g = jax.lax.dot_general(
            xi, xk, (((1,), (1,)), ((), ())),
            preferred_element_type=jnp.float32)
        d = jnp.maximum(ni + nk - 2.0 * g, 1e-37)
        eq = li == lrk
        sqrt_d = d * jax.lax.rsqrt(d)
        neg = jnp.maximum(MARGIN - sqrt_d, 0.0)
        acc_ref[...] += jnp.where(eq, d, neg * neg)
        _ = ni

    for i in range(nb - 1):
        @pl.when(i < k)
        def _off(i=i):
            xi = xs_ref[pl.ds(i * BLK, BLK), :]
            ni = jnp.transpose(
                nrow_ref[:, pl.ds(i * BLK, BLK)], (1, 0))
            li = lc_ref[pl.ds(i * BLK, BLK), :]
            tile(xi, ni, li, acco_ref)

    tile(xk, jnp.transpose(nk, (1, 0)), lc_ref[pl.ds(k * BLK, BLK), :],
         accd_ref)

    @pl.when(k == nb - 1)
    def _finish():
        total = jnp.sum(accd_ref[...] + 2.0 * acco_ref[...])
        scale = 1.0 / (n_total * (n_total - 1.0))
        out_ref[0] = total * scale

    cp.wait()


def kernel(embeddings_t, target_t):
    n, d = embeddings_t.shape
    lc = target_t.reshape(n, 1)
    lr = target_t.reshape(1, n)
    out = pl.pallas_call(
        _loss_body,
        grid=(n // BLK,),
        in_specs=[
            pl.BlockSpec((BLK, d), lambda k: (k, 0)),
            pl.BlockSpec((n, 1), lambda k: (0, 0)),
            pl.BlockSpec((1, n), lambda k: (0, 0)),
        ],
        out_specs=pl.BlockSpec(memory_space=pltpu.SMEM),
        out_shape=jax.ShapeDtypeStruct((1,), jnp.float32),
        scratch_shapes=[
            pltpu.VMEM((n, d), jnp.float32),
            pltpu.VMEM((1, n), jnp.float32),
            pltpu.VMEM((BLK, BLK), jnp.float32),
            pltpu.VMEM((BLK, BLK), jnp.float32),
            pltpu.SemaphoreType.DMA,
        ],
    )(embeddings_t, lc, lr)
    return out[0]


# speedup vs baseline: 1.5225x; 1.5225x over previous
import jax
import jax.numpy as jnp
from jax.experimental import pallas as pl
from jax.experimental.pallas import tpu as pltpu

MARGIN = 1.0
BLK = 256


def _loss_body(x_ref, lc_ref, lr_ref, out_ref):
    x = x_ref[...]
    n_total, dim = x.shape
    nb = n_total // BLK
    ncol = jnp.sum(x * x, axis=1, keepdims=True)
    one = jnp.ones((n_total, 1), jnp.float32)
    a = jnp.concatenate([-2.0 * x, ncol, one], axis=1)
    b = jnp.concatenate([x, one, ncol], axis=1)

    acc_diag = jnp.zeros((BLK, BLK), jnp.float32)
    acc_off = jnp.zeros((BLK, BLK), jnp.float32)
    for i in range(nb):
        ai = a[i * BLK:(i + 1) * BLK, :]
        li = lc_ref[pl.ds(i * BLK, BLK), :]
        for j in range(i, nb):
            bj = b[j * BLK:(j + 1) * BLK, :]
            d = jax.lax.dot_general(
                ai, bj, (((1,), (1,)), ((), ())),
                preferred_element_type=jnp.float32)
            d = jnp.maximum(d, 1e-37)
            eq = li == lr_ref[:, pl.ds(j * BLK, BLK)]
            sqrt_d = d * jax.lax.rsqrt(d)
            neg = jnp.maximum(MARGIN - sqrt_d, 0.0)
            loss = jnp.where(eq, d, neg * neg)
            if i == j:
                acc_diag = acc_diag + loss
            else:
                acc_off = acc_off + loss
    total = jnp.sum(acc_diag + 2.0 * acc_off)
    scale = 1.0 / (n_total * (n_total - 1.0))
    out_ref[0] = total * scale


def kernel(embeddings_t, target_t):
    n, _ = embeddings_t.shape
    lc = target_t.reshape(n, 1)
    lr = target_t.reshape(1, n)
    out = pl.pallas_call(
        _loss_body,
        out_specs=pl.BlockSpec(memory_space=pltpu.SMEM),
        out_shape=jax.ShapeDtypeStruct((1,), jnp.float32),
    )(embeddings_t, lc, lr)
    return out[0]


# speedup vs baseline: 1.5441x; 1.0142x over previous
import jax
import jax.numpy as jnp
from jax.experimental import pallas as pl
from jax.experimental.pallas import tpu as pltpu

MARGIN = 1.0
BLK = 256


def _loss_body(x_ref, lc_ref, lr_ref, out_ref):
    x = x_ref[...]
    n_total, dim = x.shape
    nb = n_total // BLK
    nrow = jax.lax.dot_general(
        jnp.ones((1, dim), jnp.float32), x * x,
        (((1,), (1,)), ((), ())),
        preferred_element_type=jnp.float32)

    acc_diag = jnp.zeros((BLK, BLK), jnp.float32)
    acc_off = jnp.zeros((BLK, BLK), jnp.float32)
    for i in range(nb):
        xi = x_ref[pl.ds(i * BLK, BLK), :]
        ni = jnp.transpose(nrow[:, i * BLK:(i + 1) * BLK], (1, 0))
        li = lc_ref[pl.ds(i * BLK, BLK), :]
        for j in range(i, nb):
            xj = x_ref[pl.ds(j * BLK, BLK), :]
            g = jax.lax.dot_general(
                xi, xj, (((1,), (1,)), ((), ())),
                preferred_element_type=jnp.float32)
            nj = nrow[:, j * BLK:(j + 1) * BLK]
            d = jnp.maximum(ni + nj - 2.0 * g, 1e-37)
            eq = li == lr_ref[:, pl.ds(j * BLK, BLK)]
            sqrt_d = d * jax.lax.rsqrt(d)
            neg = jnp.maximum(MARGIN - sqrt_d, 0.0)
            loss = jnp.where(eq, d, neg * neg)
            if i == j:
                acc_diag = acc_diag + loss
            else:
                acc_off = acc_off + loss
    total = jnp.sum(acc_diag + 2.0 * acc_off)
    scale = 1.0 / (n_total * (n_total - 1.0))
    out_ref[0] = total * scale


def kernel(embeddings_t, target_t):
    n, _ = embeddings_t.shape
    lc = target_t.reshape(n, 1)
    lr = target_t.reshape(1, n)
    out = pl.pallas_call(
        _loss_body,
        out_specs=pl.BlockSpec(memory_space=pltpu.SMEM),
        out_shape=jax.ShapeDtypeStruct((1,), jnp.float32),
    )(embeddings_t, lc, lr)
    return out[0]
